# Initial kernel scaffold; baseline (speedup 1.0000x reference)
import jax
import jax.numpy as jnp
from jax.experimental import pallas as pl


def kernel(params, iid, edge_index_mg, edge_index_sg, segment_ids, last_nodes, rf_feat_idx):
    raise NotImplementedError("write your pallas kernel here")


# speedup vs baseline: 1.6920x; 1.6920x over previous
import functools
import numpy as np
import jax
import jax.numpy as jnp
from jax.experimental import pallas as pl
from jax.experimental.pallas import tpu as pltpu

N_NODES = 10000
N_GRAPHS = 512
NUM_ITEMS = 100000
D = 128
NUM_TREES = 16
TREE_DEPTH = 5
NUM_LEAVES = 32
MAX_DEG = 8
EPS = 1e-5
PI_ROWS = NUM_TREES * NUM_LEAVES


def _bn(x, g, b):
    m = x.mean(axis=0)
    v = x.var(axis=0)
    return g * (x - m) / jnp.sqrt(v + EPS) + b


def _prelu(x, a):
    return jnp.where(x > 0, x, a * x)


def _seg_softmax(e, seg, n):
    mx = jax.ops.segment_max(e, seg, num_segments=n)
    mx = jnp.where(jnp.isfinite(mx), mx, 0.0)
    ex = jnp.exp(e - mx[seg])
    s = jax.ops.segment_sum(ex, seg, num_segments=n)
    return ex / jnp.maximum(s[seg], 1e-12)


def _gru_mailbox(feat, src, dst, p):
    E = src.shape[0]
    order = jnp.argsort(dst)
    dst_s = dst[order]
    src_s = src[order]
    starts = jnp.searchsorted(dst_s, jnp.arange(N_NODES))
    rank = jnp.arange(E) - starts[dst_s]
    mailbox = jnp.zeros((N_NODES, MAX_DEG, D), dtype=jnp.float32).at[dst_s, rank].set(feat[src_s])
    mask = jnp.zeros((N_NODES, MAX_DEG), dtype=jnp.float32).at[dst_s, rank].set(1.0)

    def cell(h, xm):
        x, m = xm
        gi = x @ p['W_ih'].T + p['b_ih']
        gh = h @ p['W_hh'].T + p['b_hh']
        ir, iz, inn = jnp.split(gi, 3, axis=1)
        hr, hz, hn = jnp.split(gh, 3, axis=1)
        r = jax.nn.sigmoid(ir + hr)
        z = jax.nn.sigmoid(iz + hz)
        ncand = jnp.tanh(inn + r * hn)
        hnew = (1.0 - z) * ncand + z * h
        mm = m[:, None]
        return mm * hnew + (1.0 - mm) * h, None

    h0 = jnp.zeros((N_NODES, D), dtype=jnp.float32)
    hT, _ = jax.lax.scan(cell, h0, (jnp.transpose(mailbox, (1, 0, 2)), mask.T))
    return hT


def _ndf_mu(x, feat_idx, Wd):
    B = x.shape[0]
    mus = []
    for t in range(NUM_TREES):
        xs = x[:, feat_idx[t]]
        d = jax.nn.sigmoid(xs @ Wd[t])
        dec = jnp.stack([d, 1.0 - d], axis=2)
        mu = jnp.ones((B, 1, 1), dtype=jnp.float32)
        begin, end = 1, 2
        for level in range(TREE_DEPTH):
            mu = jnp.reshape(mu, (B, -1, 1))
            mu = jnp.tile(mu, (1, 1, 2))
            mu = mu * dec[:, begin:end, :]
            begin = end
            end = begin + 2 ** (level + 1)
        mus.append(mu.reshape(B, NUM_LEAVES))
    return jnp.stack(mus, axis=0)


STATS_T = 2048
OUT_T = 1024


def _stats_body(pi_ref, m_ref, s_ref):
    j = pl.program_id(0)
    col0 = j * STATS_T
    idx = jax.lax.broadcasted_iota(jnp.int32, pi_ref.shape, 1) + col0
    x = jnp.where(idx < NUM_ITEMS, pi_ref[...], -jnp.inf)
    tile_m = jnp.max(x, axis=1, keepdims=True)

    @pl.when(j == 0)
    def _():
        m_ref[...] = jnp.full_like(m_ref, -jnp.inf)
        s_ref[...] = jnp.zeros_like(s_ref)

    m_old = m_ref[...]
    m_new = jnp.maximum(m_old, tile_m)
    t_s = jnp.sum(jnp.exp(x - m_new), axis=1, keepdims=True)
    s_ref[...] = s_ref[...] * jnp.exp(m_old - m_new) + t_s
    m_ref[...] = m_new


def _pi_stats(pi_r):
    grid = (pl.cdiv(NUM_ITEMS, STATS_T),)
    return pl.pallas_call(
        _stats_body,
        grid=grid,
        in_specs=[pl.BlockSpec((PI_ROWS, STATS_T), lambda j: (0, j))],
        out_specs=[
            pl.BlockSpec((PI_ROWS, 1), lambda j: (0, 0)),
            pl.BlockSpec((PI_ROWS, 1), lambda j: (0, 0)),
        ],
        out_shape=[
            jax.ShapeDtypeStruct((PI_ROWS, 1), jnp.float32),
            jax.ShapeDtypeStruct((PI_ROWS, 1), jnp.float32),
        ],
    )(pi_r)


def _logits_body(w_ref, sr2_ref, m_ref, pi_ref, emb_ref, out_ref):
    e = emb_ref[...]
    nrm = jnp.sqrt(jnp.sum(e * e, axis=1, keepdims=True))
    scale = jnp.minimum(1.0, 1.0 / jnp.maximum(nrm, 1e-12))
    en = e * scale
    expp = jnp.exp(pi_ref[...] - m_ref[...])
    acc = jax.lax.dot_general(
        sr2_ref[...], en, (((1,), (1,)), ((), ())),
        preferred_element_type=jnp.float32)
    acc = acc + jax.lax.dot(w_ref[...], expp, preferred_element_type=jnp.float32)
    out_ref[...] = acc


def _fused_logits(w, sr2h, m, pi_r, emb):
    grid = (pl.cdiv(NUM_ITEMS, OUT_T),)
    return pl.pallas_call(
        _logits_body,
        grid=grid,
        in_specs=[
            pl.BlockSpec((PI_ROWS, PI_ROWS), lambda j: (0, 0)),
            pl.BlockSpec((N_GRAPHS, D), lambda j: (0, 0)),
            pl.BlockSpec((PI_ROWS, 1), lambda j: (0, 0)),
            pl.BlockSpec((PI_ROWS, OUT_T), lambda j: (0, j)),
            pl.BlockSpec((OUT_T, D), lambda j: (j, 0)),
        ],
        out_specs=pl.BlockSpec((N_GRAPHS, OUT_T), lambda j: (0, j)),
        out_shape=jax.ShapeDtypeStruct((N_GRAPHS, NUM_ITEMS), jnp.float32),
    )(w, sr2h, m, pi_r, emb)


def kernel(params, iid, edge_index_mg, edge_index_sg, segment_ids, last_nodes, rf_feat_idx):
    p = params
    emb = p['emb']
    fe = emb[iid]
    fn = jnp.linalg.norm(fe, axis=-1, keepdims=True)
    feat = fe * jnp.minimum(1.0, 1.0 / jnp.maximum(fn, 1e-12))

    h = _bn(feat, p['bn0_g'], p['bn0_b'])
    neigh = _gru_mailbox(h, edge_index_mg[0], edge_index_mg[1], p)
    out = h @ p['fc_self'].T + neigh @ p['fc_neigh'].T
    out = _prelu(out, p['prelu0'])
    feat = jnp.concatenate([out, feat], axis=1)

    h = _bn(feat, p['bn1_g'], p['bn1_b'])
    q = h @ p['Wq'].T + p['bq']
    k = h @ p['Wk'].T
    v = h @ p['Wv'].T
    src, dst = edge_index_sg[0], edge_index_sg[1]
    e = jax.nn.sigmoid(q[src] + k[dst]) @ p['We_sg'].T
    a = _seg_softmax(e[:, 0], dst, N_NODES)[:, None]
    out = jax.ops.segment_sum(v[src] * a, dst, num_segments=N_NODES)
    out = _prelu(out, p['prelu1'])
    feat = jnp.concatenate([out, feat], axis=1)

    feat = jnp.concatenate([feat, jnp.zeros((feat.shape[0], D), jnp.float32)], axis=1)

    hr = _bn(feat, p['bnr_g'], p['bnr_b'])
    fu = hr @ p['Wu'].T
    fv = (hr[last_nodes] @ p['Wv_r'].T + p['bv_r'])[segment_ids]
    er = jax.nn.sigmoid(fu + fv) @ p['We_r'].T
    alpha = _seg_softmax(er[:, 0], segment_ids, N_GRAPHS)[:, None]
    rst = jax.ops.segment_sum(hr * alpha, segment_ids, num_segments=N_GRAPHS)
    sr_g = _prelu(rst @ p['Wout_r'].T, p['prelu_r'])
    sr_l = feat[last_nodes]
    sr = jnp.concatenate([sr_l, sr_g], axis=1)

    mu = _ndf_mu(sr, rf_feat_idx, p['rf_Wd'])

    srn = _bn(sr, p['bnf_g'], p['bnf_b'])
    sr2h = 0.5 * (srn @ p['fc_sr'].T)

    pi_r = p['rf_pi'].reshape(PI_ROWS, NUM_ITEMS)
    m, s = _pi_stats(pi_r)
    w = jnp.transpose(mu, (1, 0, 2)).reshape(N_GRAPHS, PI_ROWS)
    w = w * (0.5 / NUM_TREES) / s[:, 0][None, :]
    return _fused_logits(w, sr2h, m, pi_r, emb)
